# baseline (speedup 1.0000x reference)
import functools

import jax
import jax.numpy as jnp
from jax import lax
from jax.experimental import pallas as pl
from jax.experimental.pallas import tpu as pltpu
from jax.experimental.pallas import tpu_sc as plsc

N_TOK = 16 * 32 * 32
D = 64
K = 1024
BLK = 2048
HALF = N_TOK // 2
HGRID = HALF // BLK

NC = 2
NS = 16
NW = NC * NS
BPW = HALF // NW
CH = 128
NCH = BPW // CH


def _tc_body(x_ref, e_ref, idx_ref, loss_ref):
    x = x_ref[...]
    e = e_ref[...]
    sim2 = lax.dot_general(
        x + x, e, (((1,), (0,)), ((), ())),
        preferred_element_type=jnp.float32,
    )
    xsq = jnp.sum(x * x, axis=1, keepdims=True)
    esq = jnp.sum(e * e, axis=0, keepdims=True)
    dist = (xsq + esq) - sim2
    minv = jnp.min(dist, axis=1, keepdims=True)
    kiota = lax.broadcasted_iota(jnp.int32, (BLK, K), 1).astype(jnp.float32)
    idxf = jnp.min(jnp.where(dist == minv, kiota, float(K)), axis=1,
                   keepdims=True)
    idx_ref[...] = idxf.astype(jnp.int32)
    part = jnp.sum(minv)
    step = pl.program_id(0)

    @pl.when(step == 0)
    def _():
        loss_ref[0, 0] = part

    @pl.when(step != 0)
    def _():
        loss_ref[0, 0] += part


def _tc_half(block_offset):
    return pl.pallas_call(
        _tc_body,
        grid=(HGRID,),
        in_specs=[
            pl.BlockSpec((BLK, D), lambda i: (i + block_offset, 0)),
            pl.BlockSpec((D, K), lambda i: (0, 0)),
        ],
        out_specs=[
            pl.BlockSpec((BLK, 1), lambda i: (i, 0)),
            pl.BlockSpec(memory_space=pltpu.SMEM),
        ],
        out_shape=[
            jax.ShapeDtypeStruct((HALF, 1), jnp.int32),
            jax.ShapeDtypeStruct((1, 1), jnp.float32),
        ],
    )


_tc_a = _tc_half(0)
_tc_b = _tc_half(HGRID)


@functools.cache
def _sc_gather_call():
    mesh = plsc.VectorSubcoreMesh(core_axis_name="c", subcore_axis_name="s")

    @functools.partial(
        pl.kernel,
        mesh=mesh,
        compiler_params=pltpu.CompilerParams(use_tc_tiling_on_sc=False),
        out_type=jax.ShapeDtypeStruct((HALF, D), jnp.float32),
        scratch_types=[
            pltpu.VMEM((NCH, CH), jnp.int32),
            pltpu.VMEM((BPW, D), jnp.float32),
            pltpu.SemaphoreType.DMA,
        ],
    )
    def _sc_gather(et_hbm, idx_hbm, out_hbm, idx_v, rows_v, sem):
        wid = lax.axis_index("s") * NC + lax.axis_index("c")
        pltpu.sync_copy(idx_hbm.at[pl.ds(wid * NCH, NCH)], idx_v)
        copies = [
            pltpu.async_copy(
                et_hbm.at[idx_v.at[j]],
                rows_v.at[pl.ds(j * CH, CH)],
                sem,
            )
            for j in range(NCH)
        ]
        for c in copies:
            c.wait()
        pltpu.sync_copy(rows_v, out_hbm.at[pl.ds(wid * BPW, BPW)])

    return _sc_gather


def kernel(x, embeddings):
    xf = x.reshape(N_TOK, D)
    et = embeddings.T
    sc = _sc_gather_call()
    idx_a, loss_a = _tc_a(xf, embeddings)
    qa = sc(et, idx_a.reshape(NW * NCH, CH))
    idx_b, loss_b = _tc_b(xf, embeddings)
    qb = sc(et, idx_b.reshape(NW * NCH, CH))
    quantized = jnp.concatenate([qa, qb], axis=0).reshape(x.shape)
    vq_loss = (loss_a[0, 0] + loss_b[0, 0]) * (1.25 / (N_TOK * D))
    return quantized, vq_loss


# speedup vs baseline: 1.1263x; 1.1263x over previous
import functools

import jax
import jax.numpy as jnp
from jax import lax
from jax.experimental import pallas as pl
from jax.experimental.pallas import tpu as pltpu
from jax.experimental.pallas import tpu_sc as plsc

N_TOK = 16 * 32 * 32
D = 64
K = 1024
BLK = 2048
GRID = N_TOK // BLK

NC = 2
NS = 16
NW = NC * NS
BPW = N_TOK // NW
CH = 128
NCH = BPW // CH


def _tc_body(x_ref, e_ref, idx_ref, loss_ref, esq_ref, kiota_ref):
    step = pl.program_id(0)

    @pl.when(step == 0)
    def _():
        e0 = e_ref[...]
        esq_ref[...] = jnp.sum(e0 * e0, axis=0, keepdims=True)
        kiota_ref[...] = lax.broadcasted_iota(
            jnp.int32, (8, K), 1).astype(jnp.float32)

    x = x_ref[...]
    e = e_ref[...]
    sim2 = lax.dot_general(
        x + x, e, (((1,), (0,)), ((), ())),
        preferred_element_type=jnp.float32,
    )
    xsq = jnp.sum(x * x, axis=1, keepdims=True)
    esq = esq_ref[...]
    dist = (xsq + esq) - sim2
    minv = jnp.min(dist, axis=1, keepdims=True)
    kiota = jnp.broadcast_to(kiota_ref[0:1, :], (BLK, K))
    idxf = jnp.min(jnp.where(dist == minv, kiota, float(K)), axis=1,
                   keepdims=True)
    idx_ref[...] = idxf.astype(jnp.int32)
    part = jnp.sum(minv)

    @pl.when(step == 0)
    def _():
        loss_ref[0, 0] = part

    @pl.when(step != 0)
    def _():
        loss_ref[0, 0] += part


_tc_call = pl.pallas_call(
    _tc_body,
    grid=(GRID,),
    in_specs=[
        pl.BlockSpec((BLK, D), lambda i: (i, 0)),
        pl.BlockSpec((D, K), lambda i: (0, 0)),
    ],
    out_specs=[
        pl.BlockSpec((BLK, 1), lambda i: (i, 0)),
        pl.BlockSpec(memory_space=pltpu.SMEM),
    ],
    out_shape=[
        jax.ShapeDtypeStruct((N_TOK, 1), jnp.int32),
        jax.ShapeDtypeStruct((1, 1), jnp.float32),
    ],
    scratch_shapes=[
        pltpu.VMEM((1, K), jnp.float32),
        pltpu.VMEM((8, K), jnp.float32),
    ],
)


@functools.cache
def _sc_gather_call():
    mesh = plsc.VectorSubcoreMesh(core_axis_name="c", subcore_axis_name="s")

    @functools.partial(
        pl.kernel,
        mesh=mesh,
        compiler_params=pltpu.CompilerParams(use_tc_tiling_on_sc=False),
        out_type=jax.ShapeDtypeStruct((N_TOK, D), jnp.float32),
        scratch_types=[
            pltpu.VMEM((NCH, CH), jnp.int32),
            pltpu.VMEM((BPW, D), jnp.float32),
            pltpu.SemaphoreType.DMA,
        ],
    )
    def _sc_gather(et_hbm, idx_hbm, out_hbm, idx_v, rows_v, sem):
        wid = lax.axis_index("s") * NC + lax.axis_index("c")
        pltpu.sync_copy(idx_hbm.at[pl.ds(wid * NCH, NCH)], idx_v)
        copies = [
            pltpu.async_copy(
                et_hbm.at[idx_v.at[j]],
                rows_v.at[pl.ds(j * CH, CH)],
                sem,
            )
            for j in range(NCH)
        ]
        for c in copies:
            c.wait()
        pltpu.sync_copy(rows_v, out_hbm.at[pl.ds(wid * BPW, BPW)])

    return _sc_gather


def kernel(x, embeddings):
    xf = x.reshape(N_TOK, D)
    idx2d, loss_sum = _tc_call(xf, embeddings)
    et = embeddings.T
    idx_rows = idx2d.reshape(NW * NCH, CH)
    qf = _sc_gather_call()(et, idx_rows)
    quantized = qf.reshape(x.shape)
    vq_loss = loss_sum[0, 0] * (1.25 / (N_TOK * D))
    return quantized, vq_loss


# speedup vs baseline: 1.2099x; 1.0742x over previous
import functools

import jax
import jax.numpy as jnp
from jax import lax
from jax.experimental import pallas as pl
from jax.experimental.pallas import tpu as pltpu
from jax.experimental.pallas import tpu_sc as plsc

N_TOK = 16 * 32 * 32
D = 64
K = 1024
BLK = 2048
GRID = N_TOK // BLK

NC = 2
NS = 16
NW = NC * NS
BPW = N_TOK // NW
CH = 128
NCH = BPW // CH


def _tc_body(x_ref, e_ref, idx_ref, loss_ref, esq_ref, kiota_ref):
    step = pl.program_id(0)

    @pl.when(step == 0)
    def _():
        e0 = e_ref[...]
        esq_ref[...] = jnp.sum(e0 * e0, axis=0, keepdims=True)
        kiota_ref[...] = lax.broadcasted_iota(
            jnp.int32, (8, K), 1).astype(jnp.float32)

    x = x_ref[...]
    e = e_ref[...]
    sim2 = lax.dot_general(
        x + x, e, (((1,), (0,)), ((), ())),
        preferred_element_type=jnp.float32,
    )
    xsq = jnp.sum(x * x, axis=1, keepdims=True)
    esq = esq_ref[...]
    dist = (xsq + esq) - sim2
    minv = jnp.min(dist, axis=1, keepdims=True)
    kiota = jnp.broadcast_to(kiota_ref[0:1, :], (BLK, K))
    idxf = jnp.min(jnp.where(dist == minv, kiota, float(K)), axis=1,
                   keepdims=True)
    idx_ref[...] = idxf.astype(jnp.int32).reshape(BLK // CH, CH)
    part = jnp.sum(minv)

    @pl.when(step == 0)
    def _():
        loss_ref[0, 0] = part

    @pl.when(step != 0)
    def _():
        loss_ref[0, 0] += part


_tc_call = pl.pallas_call(
    _tc_body,
    grid=(GRID,),
    in_specs=[
        pl.BlockSpec((BLK, D), lambda i: (i, 0)),
        pl.BlockSpec((D, K), lambda i: (0, 0)),
    ],
    out_specs=[
        pl.BlockSpec((BLK // CH, CH), lambda i: (i, 0)),
        pl.BlockSpec(memory_space=pltpu.SMEM),
    ],
    out_shape=[
        jax.ShapeDtypeStruct((N_TOK // CH, CH), jnp.int32),
        jax.ShapeDtypeStruct((1, 1), jnp.float32),
    ],
    scratch_shapes=[
        pltpu.VMEM((1, K), jnp.float32),
        pltpu.VMEM((8, K), jnp.float32),
    ],
)


@functools.cache
def _sc_gather_call():
    mesh = plsc.VectorSubcoreMesh(core_axis_name="c", subcore_axis_name="s")

    @functools.partial(
        pl.kernel,
        mesh=mesh,
        compiler_params=pltpu.CompilerParams(use_tc_tiling_on_sc=False),
        out_type=jax.ShapeDtypeStruct((N_TOK, D), jnp.float32),
        scratch_types=[
            pltpu.VMEM((NCH, CH), jnp.int32),
            pltpu.VMEM((BPW, D), jnp.float32),
            pltpu.SemaphoreType.DMA,
        ],
    )
    def _sc_gather(et_hbm, idx_hbm, out_hbm, idx_v, rows_v, sem):
        wid = lax.axis_index("s") * NC + lax.axis_index("c")
        pltpu.sync_copy(idx_hbm.at[pl.ds(wid * NCH, NCH)], idx_v)
        copies = [
            pltpu.async_copy(
                et_hbm.at[idx_v.at[j]],
                rows_v.at[pl.ds(j * CH, CH)],
                sem,
            )
            for j in range(NCH)
        ]
        for c in copies:
            c.wait()
        pltpu.sync_copy(rows_v, out_hbm.at[pl.ds(wid * BPW, BPW)])

    return _sc_gather


def kernel(x, embeddings):
    xf = x.reshape(N_TOK, D)
    idx_rows, loss_sum = _tc_call(xf, embeddings)
    et = embeddings.T
    qf = _sc_gather_call()(et, idx_rows)
    quantized = qf.reshape(x.shape)
    vq_loss = loss_sum[0, 0] * (1.25 / (N_TOK * D))
    return quantized, vq_loss


# speedup vs baseline: 1.2228x; 1.0107x over previous
import functools

import jax
import jax.numpy as jnp
from jax import lax
from jax.experimental import pallas as pl
from jax.experimental.pallas import tpu as pltpu
from jax.experimental.pallas import tpu_sc as plsc

N_TOK = 16 * 32 * 32
D = 64
K = 1024
BLK = 4096
GRID = N_TOK // BLK

NC = 2
NS = 16
NW = NC * NS
BPW = N_TOK // NW
CH = 128
NCH = BPW // CH


def _tc_body(x_ref, e_ref, idx_ref, loss_ref, esq_ref, kiota_ref):
    step = pl.program_id(0)

    @pl.when(step == 0)
    def _():
        e0 = e_ref[...]
        esq_ref[...] = jnp.sum(e0 * e0, axis=0, keepdims=True)
        kiota_ref[...] = lax.broadcasted_iota(
            jnp.int32, (8, K), 1).astype(jnp.float32)

    x = x_ref[...]
    e = e_ref[...]
    sim2 = lax.dot_general(
        x + x, e, (((1,), (0,)), ((), ())),
        preferred_element_type=jnp.float32,
    )
    xsq = jnp.sum(x * x, axis=1, keepdims=True)
    esq = esq_ref[...]
    dist = (xsq + esq) - sim2
    minv = jnp.min(dist, axis=1, keepdims=True)
    kiota = jnp.broadcast_to(kiota_ref[0:1, :], (BLK, K))
    idxf = jnp.min(jnp.where(dist == minv, kiota, float(K)), axis=1,
                   keepdims=True)
    idx_ref[...] = idxf.astype(jnp.int32).reshape(BLK // CH, CH)
    part = jnp.sum(minv)

    @pl.when(step == 0)
    def _():
        loss_ref[0, 0] = part

    @pl.when(step != 0)
    def _():
        loss_ref[0, 0] += part


_tc_call = pl.pallas_call(
    _tc_body,
    grid=(GRID,),
    in_specs=[
        pl.BlockSpec((BLK, D), lambda i: (i, 0)),
        pl.BlockSpec((D, K), lambda i: (0, 0)),
    ],
    out_specs=[
        pl.BlockSpec((BLK // CH, CH), lambda i: (i, 0)),
        pl.BlockSpec(memory_space=pltpu.SMEM),
    ],
    out_shape=[
        jax.ShapeDtypeStruct((N_TOK // CH, CH), jnp.int32),
        jax.ShapeDtypeStruct((1, 1), jnp.float32),
    ],
    scratch_shapes=[
        pltpu.VMEM((1, K), jnp.float32),
        pltpu.VMEM((8, K), jnp.float32),
    ],
)


@functools.cache
def _sc_gather_call():
    mesh = plsc.VectorSubcoreMesh(core_axis_name="c", subcore_axis_name="s")

    @functools.partial(
        pl.kernel,
        mesh=mesh,
        compiler_params=pltpu.CompilerParams(use_tc_tiling_on_sc=False),
        out_type=jax.ShapeDtypeStruct((N_TOK, D), jnp.float32),
        scratch_types=[
            pltpu.VMEM((NCH, CH), jnp.int32),
            pltpu.VMEM((BPW, D), jnp.float32),
            pltpu.SemaphoreType.DMA,
        ],
    )
    def _sc_gather(et_hbm, idx_hbm, out_hbm, idx_v, rows_v, sem):
        wid = lax.axis_index("s") * NC + lax.axis_index("c")
        pltpu.sync_copy(idx_hbm.at[pl.ds(wid * NCH, NCH)], idx_v)
        copies = [
            pltpu.async_copy(
                et_hbm.at[idx_v.at[j]],
                rows_v.at[pl.ds(j * CH, CH)],
                sem,
            )
            for j in range(NCH)
        ]
        for c in copies:
            c.wait()
        pltpu.sync_copy(rows_v, out_hbm.at[pl.ds(wid * BPW, BPW)])

    return _sc_gather


def kernel(x, embeddings):
    xf = x.reshape(N_TOK, D)
    idx_rows, loss_sum = _tc_call(xf, embeddings)
    et = embeddings.T
    qf = _sc_gather_call()(et, idx_rows)
    quantized = qf.reshape(x.shape)
    vq_loss = loss_sum[0, 0] * (1.25 / (N_TOK * D))
    return quantized, vq_loss


# speedup vs baseline: 1.2578x; 1.0286x over previous
import functools

import jax
import jax.numpy as jnp
from jax import lax
from jax.experimental import pallas as pl
from jax.experimental.pallas import tpu as pltpu
from jax.experimental.pallas import tpu_sc as plsc

N_TOK = 16 * 32 * 32
D = 64
K = 1024
BLK = 8192
GRID = N_TOK // BLK

NC = 2
NS = 16
NW = NC * NS
BPW = N_TOK // NW
CH = 128
NCH = BPW // CH


def _tc_body(x_ref, e_ref, idx_ref, loss_ref, esq_ref, kiota_ref):
    step = pl.program_id(0)

    @pl.when(step == 0)
    def _():
        e0 = e_ref[...]
        esq_ref[...] = jnp.sum(e0 * e0, axis=0, keepdims=True)
        kiota_ref[...] = lax.broadcasted_iota(
            jnp.int32, (8, K), 1).astype(jnp.float32)

    x = x_ref[...]
    e = e_ref[...]
    sim2 = lax.dot_general(
        x + x, e, (((1,), (0,)), ((), ())),
        preferred_element_type=jnp.float32,
    )
    xsq = jnp.sum(x * x, axis=1, keepdims=True)
    esq = esq_ref[...]
    dist = (xsq + esq) - sim2
    minv = jnp.min(dist, axis=1, keepdims=True)
    kiota = jnp.broadcast_to(kiota_ref[0:1, :], (BLK, K))
    idxf = jnp.min(jnp.where(dist == minv, kiota, float(K)), axis=1,
                   keepdims=True)
    idx_ref[...] = idxf.astype(jnp.int32).reshape(BLK // CH, CH)
    part = jnp.sum(minv)

    @pl.when(step == 0)
    def _():
        loss_ref[0, 0] = part

    @pl.when(step != 0)
    def _():
        loss_ref[0, 0] += part


_tc_call = pl.pallas_call(
    _tc_body,
    grid=(GRID,),
    in_specs=[
        pl.BlockSpec((BLK, D), lambda i: (i, 0)),
        pl.BlockSpec((D, K), lambda i: (0, 0)),
    ],
    out_specs=[
        pl.BlockSpec((BLK // CH, CH), lambda i: (i, 0)),
        pl.BlockSpec(memory_space=pltpu.SMEM),
    ],
    out_shape=[
        jax.ShapeDtypeStruct((N_TOK // CH, CH), jnp.int32),
        jax.ShapeDtypeStruct((1, 1), jnp.float32),
    ],
    scratch_shapes=[
        pltpu.VMEM((1, K), jnp.float32),
        pltpu.VMEM((8, K), jnp.float32),
    ],
)


@functools.cache
def _sc_gather_call():
    mesh = plsc.VectorSubcoreMesh(core_axis_name="c", subcore_axis_name="s")

    @functools.partial(
        pl.kernel,
        mesh=mesh,
        compiler_params=pltpu.CompilerParams(use_tc_tiling_on_sc=False),
        out_type=jax.ShapeDtypeStruct((N_TOK, D), jnp.float32),
        scratch_types=[
            pltpu.VMEM((NCH, CH), jnp.int32),
            pltpu.VMEM((BPW, D), jnp.float32),
            pltpu.SemaphoreType.DMA,
        ],
    )
    def _sc_gather(et_hbm, idx_hbm, out_hbm, idx_v, rows_v, sem):
        wid = lax.axis_index("s") * NC + lax.axis_index("c")
        pltpu.sync_copy(idx_hbm.at[pl.ds(wid * NCH, NCH)], idx_v)
        copies = [
            pltpu.async_copy(
                et_hbm.at[idx_v.at[j]],
                rows_v.at[pl.ds(j * CH, CH)],
                sem,
            )
            for j in range(NCH)
        ]
        for c in copies:
            c.wait()
        pltpu.sync_copy(rows_v, out_hbm.at[pl.ds(wid * BPW, BPW)])

    return _sc_gather


def kernel(x, embeddings):
    xf = x.reshape(N_TOK, D)
    idx_rows, loss_sum = _tc_call(xf, embeddings)
    et = embeddings.T
    qf = _sc_gather_call()(et, idx_rows)
    quantized = qf.reshape(x.shape)
    vq_loss = loss_sum[0, 0] * (1.25 / (N_TOK * D))
    return quantized, vq_loss


# speedup vs baseline: 1.2748x; 1.0135x over previous
import functools

import jax
import jax.numpy as jnp
from jax import lax
from jax.experimental import pallas as pl
from jax.experimental.pallas import tpu as pltpu
from jax.experimental.pallas import tpu_sc as plsc

N_TOK = 16 * 32 * 32
D = 64
K = 1024
BLK = 8192
GRID = N_TOK // BLK

NC = 2
NS = 16
NW = NC * NS
BPW = N_TOK // NW
CH = 128
NCH = BPW // CH


def _tc_body(x_ref, e_ref, idx_ref, loss_ref, esq_ref, kiota_ref):
    step = pl.program_id(0)

    @pl.when(step == 0)
    def _():
        e0 = e_ref[...]
        esq_ref[...] = jnp.sum(e0 * e0, axis=0, keepdims=True)
        kiota_ref[...] = lax.broadcasted_iota(
            jnp.int32, (8, K), 1).astype(jnp.float32)

    x = x_ref[...]
    e = e_ref[...]
    sim2 = lax.dot_general(
        x + x, e, (((1,), (0,)), ((), ())),
        preferred_element_type=jnp.float32,
    )
    xsq = jnp.sum(x * x, axis=1, keepdims=True)
    esq = esq_ref[...]
    dist = (xsq + esq) - sim2
    minv = jnp.min(dist, axis=1, keepdims=True)
    kiota = jnp.broadcast_to(kiota_ref[0:1, :], (BLK, K))
    idxf = jnp.min(jnp.where(dist == minv, kiota, float(K)), axis=1,
                   keepdims=True)
    idx_ref[...] = idxf.astype(jnp.int32).reshape(BLK // CH, CH)
    part = jnp.sum(minv)

    @pl.when(step == 0)
    def _():
        loss_ref[0, 0] = part

    @pl.when(step != 0)
    def _():
        loss_ref[0, 0] += part

    @pl.when(step == GRID - 1)
    def _():
        loss_ref[0, 0] *= 1.25 / (N_TOK * D)


_tc_call = pl.pallas_call(
    _tc_body,
    grid=(GRID,),
    in_specs=[
        pl.BlockSpec((BLK, D), lambda i: (i, 0)),
        pl.BlockSpec((D, K), lambda i: (0, 0)),
    ],
    out_specs=[
        pl.BlockSpec((BLK // CH, CH), lambda i: (i, 0)),
        pl.BlockSpec(memory_space=pltpu.SMEM),
    ],
    out_shape=[
        jax.ShapeDtypeStruct((N_TOK // CH, CH), jnp.int32),
        jax.ShapeDtypeStruct((1, 1), jnp.float32),
    ],
    scratch_shapes=[
        pltpu.VMEM((1, K), jnp.float32),
        pltpu.VMEM((8, K), jnp.float32),
    ],
)


@functools.cache
def _sc_gather_call():
    mesh = plsc.VectorSubcoreMesh(core_axis_name="c", subcore_axis_name="s")

    @functools.partial(
        pl.kernel,
        mesh=mesh,
        compiler_params=pltpu.CompilerParams(use_tc_tiling_on_sc=False),
        out_type=jax.ShapeDtypeStruct((N_TOK, D), jnp.float32),
        scratch_types=[
            pltpu.VMEM((NCH, CH), jnp.int32),
            pltpu.VMEM((BPW, D), jnp.float32),
            pltpu.SemaphoreType.DMA,
        ],
    )
    def _sc_gather(et_hbm, idx_hbm, out_hbm, idx_v, rows_v, sem):
        wid = lax.axis_index("s") * NC + lax.axis_index("c")
        pltpu.sync_copy(idx_hbm.at[pl.ds(wid * NCH, NCH)], idx_v)
        copies = [
            pltpu.async_copy(
                et_hbm.at[idx_v.at[j]],
                rows_v.at[pl.ds(j * CH, CH)],
                sem,
            )
            for j in range(NCH)
        ]
        for c in copies:
            c.wait()
        pltpu.sync_copy(rows_v, out_hbm.at[pl.ds(wid * BPW, BPW)])

    return _sc_gather


def kernel(x, embeddings):
    xf = x.reshape(N_TOK, D)
    idx_rows, loss_sum = _tc_call(xf, embeddings)
    et = embeddings.T
    qf = _sc_gather_call()(et, idx_rows)
    quantized = qf.reshape(x.shape)
    vq_loss = loss_sum[0, 0]
    return quantized, vq_loss


# speedup vs baseline: 1.2980x; 1.0182x over previous
import functools

import jax
import jax.numpy as jnp
from jax import lax
from jax.experimental import pallas as pl
from jax.experimental.pallas import tpu as pltpu
from jax.experimental.pallas import tpu_sc as plsc

N_TOK = 16 * 32 * 32
D = 64
K = 1024
BLK = 8192
GRID = N_TOK // BLK

NC = 1
NS = 16
NW = NC * NS
BPW = N_TOK // NW
CH = 128
NCH = BPW // CH


def _tc_body(x_ref, e_ref, idx_ref, loss_ref, esq_ref, kiota_ref):
    step = pl.program_id(0)

    @pl.when(step == 0)
    def _():
        e0 = e_ref[...]
        esq_ref[...] = jnp.sum(e0 * e0, axis=0, keepdims=True)
        kiota_ref[...] = lax.broadcasted_iota(
            jnp.int32, (8, K), 1).astype(jnp.float32)

    x = x_ref[...]
    e = e_ref[...]
    sim2 = lax.dot_general(
        x + x, e, (((1,), (0,)), ((), ())),
        preferred_element_type=jnp.float32,
    )
    xsq = jnp.sum(x * x, axis=1, keepdims=True)
    esq = esq_ref[...]
    dist = (xsq + esq) - sim2
    minv = jnp.min(dist, axis=1, keepdims=True)
    kiota = jnp.broadcast_to(kiota_ref[0:1, :], (BLK, K))
    idxf = jnp.min(jnp.where(dist == minv, kiota, float(K)), axis=1,
                   keepdims=True)
    idx_ref[...] = idxf.astype(jnp.int32).reshape(BLK // CH, CH)
    part = jnp.sum(minv)

    @pl.when(step == 0)
    def _():
        loss_ref[0, 0] = part

    @pl.when(step != 0)
    def _():
        loss_ref[0, 0] += part

    @pl.when(step == GRID - 1)
    def _():
        loss_ref[0, 0] *= 1.25 / (N_TOK * D)


_tc_call = pl.pallas_call(
    _tc_body,
    grid=(GRID,),
    in_specs=[
        pl.BlockSpec((BLK, D), lambda i: (i, 0)),
        pl.BlockSpec((D, K), lambda i: (0, 0)),
    ],
    out_specs=[
        pl.BlockSpec((BLK // CH, CH), lambda i: (i, 0)),
        pl.BlockSpec(memory_space=pltpu.SMEM),
    ],
    out_shape=[
        jax.ShapeDtypeStruct((N_TOK // CH, CH), jnp.int32),
        jax.ShapeDtypeStruct((1, 1), jnp.float32),
    ],
    scratch_shapes=[
        pltpu.VMEM((1, K), jnp.float32),
        pltpu.VMEM((8, K), jnp.float32),
    ],
)


@functools.cache
def _sc_gather_call():
    mesh = plsc.VectorSubcoreMesh(core_axis_name="c", subcore_axis_name="s", num_cores=1)

    @functools.partial(
        pl.kernel,
        mesh=mesh,
        compiler_params=pltpu.CompilerParams(use_tc_tiling_on_sc=False),
        out_type=jax.ShapeDtypeStruct((N_TOK, D), jnp.float32),
        scratch_types=[
            pltpu.VMEM((NCH, CH), jnp.int32),
            pltpu.VMEM((BPW, D), jnp.float32),
            pltpu.SemaphoreType.DMA,
        ],
    )
    def _sc_gather(et_hbm, idx_hbm, out_hbm, idx_v, rows_v, sem):
        wid = lax.axis_index("s") * NC + lax.axis_index("c")
        pltpu.sync_copy(idx_hbm.at[pl.ds(wid * NCH, NCH)], idx_v)
        copies = [
            pltpu.async_copy(
                et_hbm.at[idx_v.at[j]],
                rows_v.at[pl.ds(j * CH, CH)],
                sem,
            )
            for j in range(NCH)
        ]
        for c in copies:
            c.wait()
        pltpu.sync_copy(rows_v, out_hbm.at[pl.ds(wid * BPW, BPW)])

    return _sc_gather


def kernel(x, embeddings):
    xf = x.reshape(N_TOK, D)
    idx_rows, loss_sum = _tc_call(xf, embeddings)
    et = embeddings.T
    qf = _sc_gather_call()(et, idx_rows)
    quantized = qf.reshape(x.shape)
    vq_loss = loss_sum[0, 0]
    return quantized, vq_loss
